# baseline (speedup 1.0000x reference)
import functools

import jax
import jax.numpy as jnp
from jax import lax
from jax.experimental import pallas as pl
from jax.experimental.pallas import tpu as pltpu
from jax.experimental.pallas import tpu_sc as plsc

_NC = 2
_NS = 16
_NW = _NC * _NS
_CE = 80


def _proj_body(x_ref, ws_ref, wd_ref, a_ref, b_ref):
    x = x_ref[...]
    a_ref[...] = jnp.dot(x, ws_ref[...],
                         preferred_element_type=jnp.float32).astype(jnp.bfloat16)
    b_ref[...] = jnp.dot(x, wd_ref[...],
                         preferred_element_type=jnp.float32).astype(jnp.bfloat16)


def _proj(x, w1s, w1d):
    n, d = x.shape
    blk = 1000
    return pl.pallas_call(
        _proj_body,
        grid=(n // blk,),
        in_specs=[
            pl.BlockSpec((blk, d), lambda i: (i, 0)),
            pl.BlockSpec((d, d), lambda i: (0, 0)),
            pl.BlockSpec((d, d), lambda i: (0, 0)),
        ],
        out_specs=[
            pl.BlockSpec((blk, d), lambda i: (i, 0)),
            pl.BlockSpec((blk, d), lambda i: (i, 0)),
        ],
        out_shape=[jax.ShapeDtypeStruct((n, d), jnp.bfloat16)] * 2,
    )(x, w1s, w1d)


_NBUF = 3


@functools.lru_cache(maxsize=None)
def _make_sc_gather(e_total, dw):
    epw = e_total // _NW
    nchunks = epw // _CE
    mesh = plsc.VectorSubcoreMesh(core_axis_name="c", subcore_axis_name="s",
                                  num_cores=_NC, num_subcores=_NS)

    @functools.partial(
        pl.kernel,
        out_type=[jax.ShapeDtypeStruct((e_total, dw), jnp.int32)] * 2,
        mesh=mesh,
        compiler_params=pltpu.CompilerParams(use_tc_tiling_on_sc=False),
        scratch_types=[
            pltpu.VMEM((nchunks, _CE), jnp.int32),
            pltpu.VMEM((nchunks, _CE), jnp.int32),
            pltpu.VMEM((_NBUF, _CE, dw), jnp.int32),
            pltpu.VMEM((_NBUF, _CE, dw), jnp.int32),
            pltpu.SemaphoreType.DMA((_NBUF,)),
            pltpu.SemaphoreType.DMA((_NBUF,)),
        ],
    )
    def sc_fn(a_hbm, b_hbm, src_hbm, dst_hbm, sa_hbm, sb_hbm,
              src_v, dst_v, sa_v, sb_v, gsem, wsem):
        wid = lax.axis_index("s") * _NC + lax.axis_index("c")
        base = wid * epw

        pltpu.sync_copy(src_hbm.at[wid], src_v)
        pltpu.sync_copy(dst_hbm.at[wid], dst_v)

        def start_gather(c, b):
            pltpu.async_copy(a_hbm.at[src_v.at[c]], sa_v.at[b], gsem.at[b])
            pltpu.async_copy(b_hbm.at[dst_v.at[c]], sb_v.at[b], gsem.at[b])

        def wait_gather(c, b):
            pltpu.make_async_copy(a_hbm.at[src_v.at[c]], sa_v.at[b],
                                  gsem.at[b]).wait()
            pltpu.make_async_copy(b_hbm.at[dst_v.at[c]], sb_v.at[b],
                                  gsem.at[b]).wait()

        def start_writeback(c, b):
            off = pl.multiple_of(base + c * _CE, 8)
            pltpu.async_copy(sa_v.at[b], sa_hbm.at[pl.ds(off, _CE)],
                             wsem.at[b])
            pltpu.async_copy(sb_v.at[b], sb_hbm.at[pl.ds(off, _CE)],
                             wsem.at[b])

        def wait_writeback(c, b):
            off = pl.multiple_of(base + c * _CE, 8)
            pltpu.make_async_copy(sa_v.at[b], sa_hbm.at[pl.ds(off, _CE)],
                                  wsem.at[b]).wait()
            pltpu.make_async_copy(sb_v.at[b], sb_hbm.at[pl.ds(off, _CE)],
                                  wsem.at[b]).wait()

        start_gather(0, 0)
        start_gather(1, 1)

        def chunk_body(c, carry):
            b = lax.rem(c, _NBUF)
            wait_gather(c, b)
            start_writeback(c, b)

            b2 = lax.rem(c + 2, _NBUF)

            @pl.when(c + 2 < nchunks)
            def _():
                @pl.when(c >= 1)
                def _():
                    wait_writeback(c - 1, b2)
                start_gather(c + 2, b2)

            return carry

        lax.fori_loop(0, nchunks, chunk_body, 0)

        for c in (nchunks - 2, nchunks - 1):
            wait_writeback(c, c % _NBUF)

    return sc_fn


def _final_body(sa_ref, sb_ref, e_ref, x_ref, w1e_ref, b1_ref, w2x_ref,
                w2m_ref, b2_ref, o_ref, *, nb, k):
    u = (sa_ref[...].astype(jnp.float32) + sb_ref[...].astype(jnp.float32)
         + jnp.dot(e_ref[...], w1e_ref[...],
                   preferred_element_type=jnp.float32) + b1_ref[...])
    mess = jnp.where(u > 0, u, jnp.exp(jnp.minimum(u, 0.0)) - 1.0)
    am = jnp.mean(mess.reshape(nb, k, u.shape[-1]), axis=1)
    o_ref[...] = (jnp.dot(x_ref[...], w2x_ref[...],
                          preferred_element_type=jnp.float32)
                  + jnp.dot(am, w2m_ref[...],
                            preferred_element_type=jnp.float32)
                  + b2_ref[...])


def _final(sa, sb, e, x, w1e, b1, w2x, w2m, b2):
    n, d = x.shape
    e_total, de = e.shape
    k = e_total // n
    dmsg = w1e.shape[1]
    dout = w2x.shape[1]
    nb = 200
    body = functools.partial(_final_body, nb=nb, k=k)
    return pl.pallas_call(
        body,
        grid=(n // nb,),
        in_specs=[
            pl.BlockSpec((nb * k, dmsg), lambda i: (i, 0)),
            pl.BlockSpec((nb * k, dmsg), lambda i: (i, 0)),
            pl.BlockSpec((nb * k, de), lambda i: (i, 0)),
            pl.BlockSpec((nb, d), lambda i: (i, 0)),
            pl.BlockSpec((de, dmsg), lambda i: (0, 0)),
            pl.BlockSpec((1, dmsg), lambda i: (0, 0)),
            pl.BlockSpec((d, dout), lambda i: (0, 0)),
            pl.BlockSpec((dmsg, dout), lambda i: (0, 0)),
            pl.BlockSpec((1, dout), lambda i: (0, 0)),
        ],
        out_specs=pl.BlockSpec((nb, dout), lambda i: (i, 0)),
        out_shape=jax.ShapeDtypeStruct((n, dout), jnp.float32),
    )(sa, sb, e, x, w1e, b1, w2x, w2m, b2)


def kernel(x, edge_index, e, W1, b1, W2, b2):
    n, d = x.shape
    e_total = edge_index.shape[1]
    w1s = W1[:d]
    w1d = W1[d:2 * d]
    w1e = W1[2 * d:]
    w2x = W2[:d]
    w2m = W2[d:]
    nchunks = e_total // _NW // _CE
    src = edge_index[0].astype(jnp.int32).reshape(_NW, nchunks, _CE)
    dst = edge_index[1].astype(jnp.int32).reshape(_NW, nchunks, _CE)

    a, b = _proj(x, w1s, w1d)
    a32 = lax.bitcast_convert_type(a.reshape(n, d // 2, 2), jnp.int32)
    b32 = lax.bitcast_convert_type(b.reshape(n, d // 2, 2), jnp.int32)
    sa32, sb32 = _make_sc_gather(e_total, d // 2)(a32, b32, src, dst)
    sa = lax.bitcast_convert_type(sa32, jnp.bfloat16).reshape(e_total, d)
    sb = lax.bitcast_convert_type(sb32, jnp.bfloat16).reshape(e_total, d)
    return _final(sa, sb, e, x, w1e, b1.reshape(1, -1), w2x, w2m,
                  b2.reshape(1, -1))


# speedup vs baseline: 3.8277x; 3.8277x over previous
import functools

import jax
import jax.numpy as jnp
from jax import lax
from jax.experimental import pallas as pl
from jax.experimental.pallas import tpu as pltpu
from jax.experimental.pallas import tpu_sc as plsc

_NC = 2
_NS = 16
_NW = _NC * _NS
_CE = 80


def _proj_body(x_ref, ws_ref, wd_ref, a_ref, b_ref):
    x = x_ref[...]
    a_ref[...] = jnp.dot(x, ws_ref[...], preferred_element_type=jnp.float32)
    b_ref[...] = jnp.dot(x, wd_ref[...], preferred_element_type=jnp.float32)


def _proj(x, w1s, w1d):
    n, d = x.shape
    blk = 1000
    return pl.pallas_call(
        _proj_body,
        grid=(n // blk,),
        in_specs=[
            pl.BlockSpec((blk, d), lambda i: (i, 0)),
            pl.BlockSpec((d, d), lambda i: (0, 0)),
            pl.BlockSpec((d, d), lambda i: (0, 0)),
        ],
        out_specs=[
            pl.BlockSpec((blk, d), lambda i: (i, 0)),
            pl.BlockSpec((blk, d), lambda i: (i, 0)),
        ],
        out_shape=[jax.ShapeDtypeStruct((n, d), jnp.float32)] * 2,
    )(x, w1s, w1d)


@functools.lru_cache(maxsize=None)
def _make_sc_gather(e_total, n_nodes, d):
    epw = e_total // _NW
    nchunks = epw // _CE
    nvec = d // 16
    mesh = plsc.VectorSubcoreMesh(core_axis_name="c", subcore_axis_name="s",
                                  num_cores=_NC, num_subcores=_NS)

    @functools.partial(
        pl.kernel,
        out_type=jax.ShapeDtypeStruct((e_total, d), jnp.float32),
        mesh=mesh,
        scratch_types=[
            pltpu.VMEM((epw,), jnp.int32),
            pltpu.VMEM((epw,), jnp.int32),
            pltpu.VMEM((_CE, d), jnp.float32),
            pltpu.VMEM((_CE, d), jnp.float32),
            pltpu.SemaphoreType.DMA,
        ],
    )
    def sc_fn(a_hbm, b_hbm, src_hbm, dst_hbm, u_hbm,
              src_v, dst_v, wa_v, wb_v, gsem):
        wid = lax.axis_index("s") * _NC + lax.axis_index("c")
        base = wid * epw

        pltpu.sync_copy(src_hbm.at[pl.ds(base, epw)], src_v)
        pltpu.sync_copy(dst_hbm.at[pl.ds(base, epw)], dst_v)

        def chunk_body(c, carry):
            io = pl.multiple_of(c * _CE, 8)
            ga = pltpu.make_async_copy(
                a_hbm.at[src_v.at[pl.ds(io, _CE)]], wa_v, gsem)
            gb = pltpu.make_async_copy(
                b_hbm.at[dst_v.at[pl.ds(io, _CE)]], wb_v, gsem)
            ga.start()
            gb.start()
            ga.wait()
            gb.wait()

            def row_body(r, rc):
                for v in range(nvec):
                    sl = pl.ds(v * 16, 16)
                    wa_v[r, sl] = wa_v[r, sl] + wb_v[r, sl]
                return rc

            lax.fori_loop(0, _CE, row_body, 0)

            off = pl.multiple_of(base + c * _CE, 8)
            pltpu.sync_copy(wa_v, u_hbm.at[pl.ds(off, _CE)])
            return carry

        lax.fori_loop(0, nchunks, chunk_body, 0)

    return sc_fn


def _final_body(u_ref, e_ref, x_ref, w1e_ref, b1_ref,
                w2x_ref, w2m_ref, b2_ref, o_ref, *, nb, k, d):
    u = (u_ref[...]
         + jnp.dot(e_ref[...], w1e_ref[...], preferred_element_type=jnp.float32)
         + b1_ref[...])
    mess = jnp.where(u > 0, u, jnp.exp(jnp.minimum(u, 0.0)) - 1.0)
    am = jnp.mean(mess.reshape(nb, k, d), axis=1)
    o_ref[...] = (jnp.dot(x_ref[...], w2x_ref[...],
                          preferred_element_type=jnp.float32)
                  + jnp.dot(am, w2m_ref[...],
                            preferred_element_type=jnp.float32)
                  + b2_ref[...])


def _final(u, e, x, w1e, b1, w2x, w2m, b2):
    n, d = x.shape
    e_total, de = e.shape
    k = e_total // n
    dout = w2x.shape[1]
    nb = 200
    body = functools.partial(_final_body, nb=nb, k=k, d=d)
    return pl.pallas_call(
        body,
        grid=(n // nb,),
        in_specs=[
            pl.BlockSpec((nb * k, d), lambda i: (i, 0)),
            pl.BlockSpec((nb * k, de), lambda i: (i, 0)),
            pl.BlockSpec((nb, d), lambda i: (i, 0)),
            pl.BlockSpec((de, d), lambda i: (0, 0)),
            pl.BlockSpec((1, d), lambda i: (0, 0)),
            pl.BlockSpec((d, dout), lambda i: (0, 0)),
            pl.BlockSpec((d, dout), lambda i: (0, 0)),
            pl.BlockSpec((1, dout), lambda i: (0, 0)),
        ],
        out_specs=pl.BlockSpec((nb, dout), lambda i: (i, 0)),
        out_shape=jax.ShapeDtypeStruct((n, dout), jnp.float32),
    )(u, e, x, w1e, b1, w2x, w2m, b2)


def kernel(x, edge_index, e, W1, b1, W2, b2):
    n, d = x.shape
    e_total = edge_index.shape[1]
    w1s = W1[:d]
    w1d = W1[d:2 * d]
    w1e = W1[2 * d:]
    w2x = W2[:d]
    w2m = W2[d:]
    src = edge_index[0].astype(jnp.int32)
    dst = edge_index[1].astype(jnp.int32)

    a, b = _proj(x, w1s, w1d)
    u = _make_sc_gather(e_total, n, d)(a, b, src, dst)
    return _final(u, e, x, w1e, b1.reshape(1, -1),
                  w2x, w2m, b2.reshape(1, -1))
